# baseline (speedup 1.0000x reference)
import jax
import jax.numpy as jnp
from jax.experimental import pallas as pl
from jax.experimental.pallas import tpu as pltpu
from jax.experimental.pallas import tpu_sc as plsc

_BATCH = 4096
_SEQ = 200
_EMB = 64
_B = _BATCH * _SEQ
_W = 128


def kernel(token_ids, table):
    idx = token_ids.reshape(1, _B).astype(jnp.int32)
    mesh = plsc.VectorSubcoreMesh(core_axis_name="core", subcore_axis_name="subcore")

    @pl.kernel(
        out_type=jax.ShapeDtypeStruct((_B, _EMB), table.dtype),
        mesh=mesh,
        compiler_params=pltpu.CompilerParams(use_tc_tiling_on_sc=False),
    )
    def k(tab_hbm, i_hbm, o_hbm):
        def body(i_vmem, o_vmem):
            pltpu.sync_copy(tab_hbm.at[i_vmem.at[0]], o_vmem)

        pltpu.emit_pipeline(
            body,
            grid=(_B // _W,),
            in_specs=[pl.BlockSpec((1, _W), index_map=lambda i: (0, i))],
            out_specs=[pl.BlockSpec((_W, _EMB), index_map=lambda i: (i, 0))],
            core_axis_name=("core", "subcore"),
            dimension_semantics=(pltpu.PARALLEL,),
        )(i_hbm, o_hbm)

    return k(table, idx).reshape(_BATCH, _SEQ, _EMB)


# speedup vs baseline: 1.0747x; 1.0747x over previous
import jax
import jax.numpy as jnp
from jax.experimental import pallas as pl
from jax.experimental.pallas import tpu as pltpu
from jax.experimental.pallas import tpu_sc as plsc

_BATCH = 4096
_SEQ = 200
_EMB = 64
_B = _BATCH * _SEQ
_NW = 32
_N_PER_W = _B // _NW
_W = 128
_NWIN = _N_PER_W // _W
_NBUF = 8


def kernel(token_ids, table):
    idx = token_ids.reshape(_NW, _NWIN, _W).astype(jnp.int32)
    mesh = plsc.VectorSubcoreMesh(core_axis_name="core", subcore_axis_name="subcore")

    @pl.kernel(
        out_type=jax.ShapeDtypeStruct((_B, _EMB), table.dtype),
        mesh=mesh,
        compiler_params=pltpu.CompilerParams(use_tc_tiling_on_sc=False),
        scratch_types=[
            pltpu.VMEM((_NWIN, _W), jnp.int32),
            pltpu.VMEM((_NBUF, _W, _EMB), jnp.float32),
            pltpu.SemaphoreType.DMA((_NBUF,)),
            pltpu.SemaphoreType.DMA((_NBUF,)),
            pltpu.SemaphoreType.DMA,
        ],
    )
    def k(tab_hbm, i_hbm, o_hbm, idx_v, bufs, gsem, osem, isem):
        wid = jax.lax.axis_index("subcore") * 2 + jax.lax.axis_index("core")
        base = wid * _N_PER_W

        pltpu.async_copy(i_hbm.at[wid], idx_v, isem).wait()

        def start_gather(win, b):
            pltpu.make_async_copy(
                tab_hbm.at[idx_v.at[win]], bufs.at[b], gsem.at[b]
            ).start()

        def drain_slot(win, b):
            pltpu.make_async_copy(
                tab_hbm.at[idx_v.at[win]], bufs.at[b], gsem.at[b]
            ).wait()
            cp = pltpu.make_async_copy(
                bufs.at[b], o_hbm.at[pl.ds(base + win * _W, _W)], osem.at[b]
            )
            cp.start()
            cp.wait()

        for b in range(_NBUF):
            start_gather(b, b)

        @pl.loop(_NBUF, _NWIN, step=_NBUF)
        def _(g0):
            for b in range(_NBUF):
                drain_slot(g0 - _NBUF + b, b)
                start_gather(g0 + b, b)

        for b in range(_NBUF):
            drain_slot(_NWIN - _NBUF + b, b)

    return k(table, idx).reshape(_BATCH, _SEQ, _EMB)


# speedup vs baseline: 1.3134x; 1.2221x over previous
import jax
import jax.numpy as jnp
from jax.experimental import pallas as pl
from jax.experimental.pallas import tpu as pltpu
from jax.experimental.pallas import tpu_sc as plsc

_BATCH = 4096
_SEQ = 200
_EMB = 64
_B = _BATCH * _SEQ
_NW = 32
_N_PER_W = _B // _NW
_W = 128
_NWIN = _N_PER_W // _W
_NBUF = 4


def kernel(token_ids, table):
    idx = token_ids.reshape(_NW, _NWIN, _W).astype(jnp.int32)
    tab128 = jnp.pad(table, ((0, 0), (0, 128 - _EMB)))
    mesh = plsc.VectorSubcoreMesh(core_axis_name="core", subcore_axis_name="subcore")

    @pl.kernel(
        out_type=jax.ShapeDtypeStruct((_B, 128), table.dtype),
        mesh=mesh,
        compiler_params=pltpu.CompilerParams(use_tc_tiling_on_sc=False),
        scratch_types=[
            pltpu.VMEM((_NWIN, _W), jnp.int32),
            pltpu.VMEM((_NBUF, _W, 128), jnp.float32),
            pltpu.SemaphoreType.DMA((_NBUF,)),
            pltpu.SemaphoreType.DMA((_NBUF,)),
            pltpu.SemaphoreType.DMA,
        ],
    )
    def k(tab_hbm, i_hbm, o_hbm, idx_v, bufs, gsem, osem, isem):
        wid = jax.lax.axis_index("subcore") * 2 + jax.lax.axis_index("core")
        base = wid * _N_PER_W

        pltpu.async_copy(i_hbm.at[wid], idx_v, isem).wait()

        def start_gather(win, b):
            pltpu.make_async_copy(
                tab_hbm.at[idx_v.at[win]], bufs.at[b], gsem.at[b]
            ).start()

        def drain_slot(win, b):
            pltpu.make_async_copy(
                tab_hbm.at[idx_v.at[win]], bufs.at[b], gsem.at[b]
            ).wait()
            cp = pltpu.make_async_copy(
                bufs.at[b], o_hbm.at[pl.ds(base + win * _W, _W)], osem.at[b]
            )
            cp.start()
            cp.wait()

        for b in range(_NBUF):
            start_gather(b, b)

        @pl.loop(_NBUF, _NWIN, step=_NBUF)
        def _(g0):
            for b in range(_NBUF):
                drain_slot(g0 - _NBUF + b, b)
                start_gather(g0 + b, b)

        for b in range(_NBUF):
            drain_slot(_NWIN - _NBUF + b, b)

    out128 = k(tab128, idx)
    return out128[:, :_EMB].reshape(_BATCH, _SEQ, _EMB)
